# baseline (speedup 1.0000x reference)
import functools

import jax
import jax.numpy as jnp
from jax import lax
from jax.experimental import pallas as pl
from jax.experimental.pallas import tpu as pltpu
from jax.experimental.pallas import tpu_sc as plsc

_DIM = 1152
_DIMP = 1280
_WPAD = _DIMP // 2
_BATCH = 16384
_ROWS_PAD = 1024
_NC = 2
_NS = 16
_NW = _NC * _NS

_B_SC = 7168
_B_TC = _BATCH - _B_SC

_CHUNK = 32
_BPW = _B_SC // _NW
_NCHUNK = _BPW // _CHUNK
_NBUF = 3

_BM = 512
_NBLK = _BATCH // _BM
_BLK_SC = _B_SC // _BM


def _make_sc_gather():
    mesh = plsc.VectorSubcoreMesh(core_axis_name="c", subcore_axis_name="s")

    @functools.partial(
        pl.kernel,
        mesh=mesh,
        out_type=jax.ShapeDtypeStruct((_B_SC, _WPAD), jnp.int32),
        scratch_types=[
            pltpu.VMEM((_BPW,), jnp.int32),
            pltpu.VMEM((_CHUNK, _WPAD), jnp.int32),
            pltpu.VMEM((_CHUNK, _WPAD), jnp.int32),
            pltpu.VMEM((_CHUNK, _WPAD), jnp.int32),
            pltpu.SemaphoreType.DMA,
            pltpu.SemaphoreType.DMA,
            pltpu.SemaphoreType.DMA,
            pltpu.SemaphoreType.DMA,
            pltpu.SemaphoreType.DMA,
            pltpu.SemaphoreType.DMA,
        ],
    )
    def k(table_hbm, idx_hbm, out_hbm, idx_v, buf0, buf1, buf2,
          gs0, gs1, gs2, ws0, ws1, ws2):
        wid = lax.axis_index("s") * _NC + lax.axis_index("c")
        base = wid * _BPW
        pltpu.sync_copy(idx_hbm.at[pl.ds(base, _BPW)], idx_v)
        bufs = (buf0, buf1, buf2)
        gsems = (gs0, gs1, gs2)
        wsems = (ws0, ws1, ws2)

        def gather_start(c):
            return pltpu.async_copy(
                table_hbm.at[idx_v.at[pl.ds(c * _CHUNK, _CHUNK)]],
                bufs[c % _NBUF], gsems[c % _NBUF])

        def write_start(c):
            return pltpu.async_copy(
                bufs[c % _NBUF], out_hbm.at[pl.ds(base + c * _CHUNK, _CHUNK)],
                wsems[c % _NBUF])

        gcp = [None] * _NCHUNK
        wcp = [None] * _NCHUNK
        for c in range(min(_NBUF, _NCHUNK)):
            gcp[c] = gather_start(c)
        for c in range(_NCHUNK):
            gcp[c].wait()
            wcp[c] = write_start(c)
            if c + _NBUF < _NCHUNK:
                wcp[c].wait()
                gcp[c + _NBUF] = gather_start(c + _NBUF)
        for c in range(max(0, _NCHUNK - _NBUF), _NCHUNK):
            wcp[c].wait()

    return k


_sc_gather = _make_sc_gather()


def _upcast_body(sc_ref, out_ref):
    x = sc_ref[...]
    lo = jax.lax.bitcast_convert_type(
        jax.lax.shift_left(x, jnp.int32(16)), jnp.float32)
    hi = jax.lax.bitcast_convert_type(
        jax.lax.bitwise_and(x, jnp.int32(-65536)), jnp.float32)
    out_ref[:, 0:_WPAD] = lo
    out_ref[:, _WPAD:_DIM] = hi[:, 0:_DIM - _WPAD]


def _tc_upcast(sc_i32):
    return pl.pallas_call(
        _upcast_body,
        grid=(_BLK_SC,),
        in_specs=[pl.BlockSpec((_BM, _WPAD), lambda i: (i, 0))],
        out_specs=pl.BlockSpec((_BM, _DIM), lambda i: (i, 0)),
        out_shape=jax.ShapeDtypeStruct((_BATCH, _DIM), jnp.float32),
    )(sc_i32)


def _mm_body(prev_ref, lab_ref, tab_ref, out_ref):
    del prev_ref
    labs = lab_ref[0]
    oh = (labs.reshape(_BM, 1) ==
          jax.lax.broadcasted_iota(jnp.int32, (_BM, _ROWS_PAD), 1))
    oh = oh.astype(jnp.bfloat16)
    out_ref[...] = jnp.dot(oh, tab_ref[...],
                           preferred_element_type=jnp.float32)


def _tc_matmul(prev, labels3, table_bf16):
    return pl.pallas_call(
        _mm_body,
        grid=(_NBLK - _BLK_SC,),
        in_specs=[
            pl.BlockSpec(memory_space=pl.ANY),
            pl.BlockSpec((1, 1, _BM), lambda i: (i, 0, 0)),
            pl.BlockSpec((_ROWS_PAD, _DIM), lambda i: (0, 0)),
        ],
        out_specs=pl.BlockSpec((_BM, _DIM), lambda i: (i + _BLK_SC, 0)),
        out_shape=jax.ShapeDtypeStruct((_BATCH, _DIM), jnp.float32),
        input_output_aliases={0: 0},
    )(prev, labels3, table_bf16)


def kernel(labels, train, embedding_table):
    del train
    labels = labels.astype(jnp.int32)
    idx_sc = labels[:_B_SC]
    labels3 = labels[_B_SC:].reshape(_NBLK - _BLK_SC, 1, _BM)
    table_bf16 = jnp.concatenate(
        [embedding_table,
         jnp.zeros((_ROWS_PAD - embedding_table.shape[0], _DIM),
                   embedding_table.dtype)], axis=0).astype(jnp.bfloat16)
    lo = table_bf16[:, :_WPAD].view(jnp.uint16).astype(jnp.uint32)
    hi = jnp.concatenate(
        [table_bf16[:, _WPAD:],
         jnp.zeros((_ROWS_PAD, _DIMP - _DIM), jnp.bfloat16)],
        axis=1).view(jnp.uint16).astype(jnp.uint32)
    table_i32 = (lo | (hi << jnp.uint32(16))).view(jnp.int32)
    sc_i32 = _sc_gather(table_i32, idx_sc)
    up = _tc_upcast(sc_i32)
    return _tc_matmul(up, labels3, table_bf16)


# speedup vs baseline: 1.0395x; 1.0395x over previous
import functools

import jax
import jax.numpy as jnp
from jax import lax
from jax.experimental import pallas as pl
from jax.experimental.pallas import tpu as pltpu
from jax.experimental.pallas import tpu_sc as plsc

_DIM = 1152
_DIMP = 1280
_WPAD = _DIMP // 2
_BATCH = 16384
_ROWS_PAD = 1024
_NC = 2
_NS = 16
_NW = _NC * _NS

_BM = 512
_NBLK = _BATCH // _BM
_U = 224
_M = _BM - _U

_B_SC = _NBLK * _U
_CHUNK = 32
_BPW = _B_SC // _NW
_NCHUNK = _BPW // _CHUNK
_NBUF = 3


def _make_sc_gather():
    mesh = plsc.VectorSubcoreMesh(core_axis_name="c", subcore_axis_name="s")

    @functools.partial(
        pl.kernel,
        mesh=mesh,
        out_type=jax.ShapeDtypeStruct((_B_SC, _WPAD), jnp.int32),
        scratch_types=[
            pltpu.VMEM((_BPW,), jnp.int32),
            pltpu.VMEM((_CHUNK, _WPAD), jnp.int32),
            pltpu.VMEM((_CHUNK, _WPAD), jnp.int32),
            pltpu.VMEM((_CHUNK, _WPAD), jnp.int32),
            pltpu.SemaphoreType.DMA,
            pltpu.SemaphoreType.DMA,
            pltpu.SemaphoreType.DMA,
            pltpu.SemaphoreType.DMA,
            pltpu.SemaphoreType.DMA,
            pltpu.SemaphoreType.DMA,
        ],
    )
    def k(table_hbm, idx_hbm, out_hbm, idx_v, buf0, buf1, buf2,
          gs0, gs1, gs2, ws0, ws1, ws2):
        wid = lax.axis_index("s") * _NC + lax.axis_index("c")
        base = wid * _BPW
        pltpu.sync_copy(idx_hbm.at[pl.ds(base, _BPW)], idx_v)
        bufs = (buf0, buf1, buf2)
        gsems = (gs0, gs1, gs2)
        wsems = (ws0, ws1, ws2)

        def gather_start(c):
            return pltpu.async_copy(
                table_hbm.at[idx_v.at[pl.ds(c * _CHUNK, _CHUNK)]],
                bufs[c % _NBUF], gsems[c % _NBUF])

        def write_start(c):
            return pltpu.async_copy(
                bufs[c % _NBUF], out_hbm.at[pl.ds(base + c * _CHUNK, _CHUNK)],
                wsems[c % _NBUF])

        gcp = [None] * _NCHUNK
        wcp = [None] * _NCHUNK
        for c in range(min(_NBUF, _NCHUNK)):
            gcp[c] = gather_start(c)
        for c in range(_NCHUNK):
            gcp[c].wait()
            wcp[c] = write_start(c)
            if c + _NBUF < _NCHUNK:
                wcp[c].wait()
                gcp[c + _NBUF] = gather_start(c + _NBUF)
        for c in range(max(0, _NCHUNK - _NBUF), _NCHUNK):
            wcp[c].wait()

    return k


_sc_gather = _make_sc_gather()


def _tc_body(sc_ref, lab_ref, tab_ref, out_ref):
    x = sc_ref[...]
    lo = jax.lax.bitcast_convert_type(
        jax.lax.shift_left(x, jnp.int32(16)), jnp.float32)
    hi = jax.lax.bitcast_convert_type(
        jax.lax.bitwise_and(x, jnp.int32(-65536)), jnp.float32)
    out_ref[0:_U, 0:_WPAD] = lo
    out_ref[0:_U, _WPAD:_DIM] = hi[:, 0:_DIM - _WPAD]
    labs = lab_ref[0]
    oh = (labs.reshape(_M, 1) ==
          jax.lax.broadcasted_iota(jnp.int32, (_M, _ROWS_PAD), 1))
    oh = oh.astype(jnp.bfloat16)
    out_ref[_U:_BM, :] = jnp.dot(oh, tab_ref[...],
                                 preferred_element_type=jnp.float32)


def _tc_assemble(sc_i32, labels_mm, table_bf16):
    return pl.pallas_call(
        _tc_body,
        grid=(_NBLK,),
        in_specs=[
            pl.BlockSpec((_U, _WPAD), lambda i: (i, 0)),
            pl.BlockSpec((1, 1, _M), lambda i: (i, 0, 0)),
            pl.BlockSpec((_ROWS_PAD, _DIM), lambda i: (0, 0)),
        ],
        out_specs=pl.BlockSpec((_BM, _DIM), lambda i: (i, 0)),
        out_shape=jax.ShapeDtypeStruct((_BATCH, _DIM), jnp.float32),
    )(sc_i32, labels_mm, table_bf16)


def kernel(labels, train, embedding_table):
    del train
    labels2 = labels.astype(jnp.int32).reshape(_NBLK, _BM)
    idx_sc = labels2[:, :_U].reshape(_B_SC)
    labels_mm = labels2[:, _U:].reshape(_NBLK, 1, _M)
    table_bf16 = jnp.concatenate(
        [embedding_table,
         jnp.zeros((_ROWS_PAD - embedding_table.shape[0], _DIM),
                   embedding_table.dtype)], axis=0).astype(jnp.bfloat16)
    lo = table_bf16[:, :_WPAD].view(jnp.uint16).astype(jnp.uint32)
    hi = jnp.concatenate(
        [table_bf16[:, _WPAD:],
         jnp.zeros((_ROWS_PAD, _DIMP - _DIM), jnp.bfloat16)],
        axis=1).view(jnp.uint16).astype(jnp.uint32)
    table_i32 = (lo | (hi << jnp.uint32(16))).view(jnp.int32)
    sc_i32 = _sc_gather(table_i32, idx_sc)
    return _tc_assemble(sc_i32, labels_mm, table_bf16)
